# baseline (speedup 1.0000x reference)
import functools
import jax
import jax.numpy as jnp
from jax import lax
from jax.experimental import pallas as pl
from jax.experimental.pallas import tpu as pltpu
from jax.experimental.pallas import tpu_sc as plsc

B = 4096
NNEG = 50
D = 32
BATA = 0.01
LAMDA = 0.01
T = float(B - 1) ** -0.5

NC = 2
NS = 16
NW = NC * NS
UPW = B // NW
GPW = UPW * NNEG // 128
NCHUNK = 5
GPC = GPW // NCHUNK
ROWS_PC = GPC * 128


@functools.cache
def _build_sc_gather():
  mesh = plsc.VectorSubcoreMesh(core_axis_name="c", subcore_axis_name="s")

  @functools.partial(
      pl.kernel,
      out_type=(
          jax.ShapeDtypeStruct((B, D), jnp.float32),
          jax.ShapeDtypeStruct((B, D), jnp.float32),
          jax.ShapeDtypeStruct((B * NNEG, D), jnp.float32),
          jax.ShapeDtypeStruct((NW, 1, UPW), jnp.float32),
          jax.ShapeDtypeStruct((NW, GPW, 128), jnp.float32),
      ),
      mesh=mesh,
      scratch_types=(
          pltpu.VMEM((1, UPW), jnp.int32),
          pltpu.VMEM((1, UPW), jnp.int32),
          pltpu.VMEM((GPW, 128), jnp.int32),
          pltpu.VMEM((UPW, D), jnp.float32),
          pltpu.VMEM((UPW, D), jnp.float32),
          pltpu.VMEM((ROWS_PC, D), jnp.float32),
          pltpu.VMEM((1, UPW), jnp.float32),
          pltpu.VMEM((GPW, 128), jnp.float32),
          pltpu.SemaphoreType.DMA,
      ),
      compiler_params=pltpu.CompilerParams(use_tc_tiling_on_sc=False),
  )
  def _sc_gather(users_hbm, pos_hbm, neg_hbm, pu_hbm, qi_hbm, bi_hbm,
                 urows_hbm, prows_hbm, nrows_hbm, bpos_hbm, bneg_hbm,
                 uidx_v, pidx_v, nidx_v, urows_v, prows_v, nrows_v,
                 bpos_v, bneg_v, sem):
    w = lax.axis_index("s") * NC + lax.axis_index("c")
    pltpu.sync_copy(users_hbm.at[w], uidx_v.at[0])
    pltpu.sync_copy(pos_hbm.at[w], pidx_v.at[0])
    pltpu.sync_copy(neg_hbm.at[w], nidx_v)
    cp_u = pltpu.async_copy(pu_hbm.at[uidx_v.at[0]], urows_v, sem)
    cp_p = pltpu.async_copy(qi_hbm.at[pidx_v.at[0]], prows_v, sem)
    cp_b = pltpu.async_copy(bi_hbm.at[pidx_v.at[0]], bpos_v.at[0], sem)
    cp_u.wait()
    cp_p.wait()
    cp_b.wait()
    pltpu.sync_copy(urows_v, urows_hbm.at[pl.ds(w * UPW, UPW)])
    pltpu.sync_copy(prows_v, prows_hbm.at[pl.ds(w * UPW, UPW)])
    pltpu.sync_copy(bpos_v, bpos_hbm.at[w])

    def chunk(c, carry):
      cps = []
      for j in range(GPC):
        g = c * GPC + j
        cps.append(pltpu.async_copy(qi_hbm.at[nidx_v.at[g]],
                                    nrows_v.at[pl.ds(j * 128, 128)], sem))
        cps.append(pltpu.async_copy(bi_hbm.at[nidx_v.at[g]], bneg_v.at[g],
                                    sem))
      for cp in cps:
        cp.wait()
      pltpu.sync_copy(
          nrows_v,
          nrows_hbm.at[pl.ds(w * (GPW * 128) + c * ROWS_PC, ROWS_PC)])
      return carry

    lax.fori_loop(0, NCHUNK, chunk, 0)
    pltpu.sync_copy(bneg_v, bneg_hbm.at[w])

  return _sc_gather


def _tc_loss_body(u_ref, p_ref, n_ref, bi_ref, bj_ref, out_ref):
  i = pl.program_id(0)

  @pl.when(i == 0)
  def _():
    out_ref[0, 0] = 0.0

  u = u_ref[...]
  p = p_ref[...]
  n = n_ref[...]
  bi = bi_ref[...]
  bj = bj_ref[...]
  posdot = jnp.sum(u * p, axis=-1)
  pos_scores = T * posdot + bi[0, 0]
  negdot = jnp.sum(u[:, None, :] * n, axis=-1)
  neg_scores = T * negdot + bj
  diff = pos_scores[:, None] - neg_scores - 1.0
  part = (jnp.sum(diff * diff) / float(B * NNEG)
          + BATA * jnp.sum(u * u)
          + BATA * (jnp.sum(p * p) + jnp.sum(n * n))
          + LAMDA * (jnp.sum(bi * bi) + jnp.sum(bj * bj)))
  out_ref[0, 0] += part


@functools.cache
def _build_tc_loss():
  return pl.pallas_call(
      _tc_loss_body,
      grid=(NW,),
      in_specs=[
          pl.BlockSpec((UPW, D), lambda i: (i, 0)),
          pl.BlockSpec((UPW, D), lambda i: (i, 0)),
          pl.BlockSpec((UPW, NNEG, D), lambda i: (i, 0, 0)),
          pl.BlockSpec((1, 1, UPW), lambda i: (i, 0, 0)),
          pl.BlockSpec((UPW, NNEG), lambda i: (i, 0)),
      ],
      out_specs=pl.BlockSpec(memory_space=pltpu.SMEM),
      out_shape=jax.ShapeDtypeStruct((1, 1), jnp.float32),
      compiler_params=pltpu.CompilerParams(dimension_semantics=("arbitrary",)),
  )


def kernel(users, pos_items, neg_items, pu, qi, bi):
  users2 = users.reshape(NW, UPW)
  pos2 = pos_items.reshape(NW, UPW)
  neg3 = neg_items.reshape(NW, GPW, 128)
  bif = bi.reshape(-1)
  urows, prows, nrows, bpos, bneg = _build_sc_gather()(
      users2, pos2, neg3, pu, qi, bif)
  res = _build_tc_loss()(urows, prows, nrows.reshape(B, NNEG, D), bpos,
                         bneg.reshape(B, NNEG))
  return res[0, 0]


# speedup vs baseline: 1.0929x; 1.0929x over previous
import functools
import jax
import jax.numpy as jnp
from jax import lax
from jax.experimental import pallas as pl
from jax.experimental.pallas import tpu as pltpu
from jax.experimental.pallas import tpu_sc as plsc

B = 4096
NNEG = 50
D = 32
BATA = 0.01
LAMDA = 0.01
T = float(B - 1) ** -0.5

NC = 2
NS = 16
NW = NC * NS
UPW = B // NW
NPW = UPW * NNEG
GPW = NPW // 128
NCHUNK = 5
GPC = GPW // NCHUNK
ROWS_PC = GPC * 128
PG_PC = ROWS_PC // 16


@functools.cache
def _build_sc_fused():
  mesh = plsc.VectorSubcoreMesh(core_axis_name="c", subcore_axis_name="s")

  @functools.partial(
      pl.kernel,
      out_type=jax.ShapeDtypeStruct((NW, 1, 16), jnp.float32),
      mesh=mesh,
      scratch_types=(
          pltpu.VMEM((1, UPW), jnp.int32),
          pltpu.VMEM((1, UPW), jnp.int32),
          pltpu.VMEM((GPW, 128), jnp.int32),
          pltpu.VMEM((UPW, D), jnp.float32),
          pltpu.VMEM((UPW, D), jnp.float32),
          pltpu.VMEM((ROWS_PC, D), jnp.float32),
          pltpu.VMEM((1, UPW), jnp.float32),
          pltpu.VMEM((ROWS_PC,), jnp.float32),
          pltpu.VMEM((1, UPW), jnp.float32),
          pltpu.VMEM((1, 16), jnp.float32),
          pltpu.SemaphoreType.DMA,
      ),
      compiler_params=pltpu.CompilerParams(
          use_tc_tiling_on_sc=False, needs_layout_passes=False),
  )
  def _sc_fused(users_hbm, pos_hbm, neg_hbm, pu_hbm, qi_hbm, bi_hbm,
                out_hbm,
                uidx_v, pidx_v, nidx_v, urows_v, prows_v, nrows_v,
                bpos_v, bneg_v, pscore_v, out_v, sem):
    w = lax.axis_index("s") * NC + lax.axis_index("c")
    pltpu.sync_copy(users_hbm.at[w], uidx_v.at[0])
    pltpu.sync_copy(pos_hbm.at[w], pidx_v.at[0])
    pltpu.sync_copy(neg_hbm.at[w], nidx_v)
    cp_u = pltpu.async_copy(pu_hbm.at[uidx_v.at[0]], urows_v, sem)
    cp_p = pltpu.async_copy(qi_hbm.at[pidx_v.at[0]], prows_v, sem)
    cp_b = pltpu.async_copy(bi_hbm.at[pidx_v.at[0]], bpos_v.at[0], sem)
    cp_u.wait()
    cp_p.wait()
    cp_b.wait()

    lanes = jnp.arange(16, dtype=jnp.int32)
    zero16 = jnp.zeros((16,), jnp.int32)
    f32z = jnp.zeros((16,), jnp.float32)

    acc_u2 = f32z
    acc_p2 = f32z
    acc_bi2 = f32z
    for g in range(UPW // 16):
      uids = lanes + (g * 16)
      pd = f32z
      for d in range(D):
        cold = jnp.full((16,), d, jnp.int32)
        uv = plsc.load_gather(urows_v, [uids, cold])
        pv = plsc.load_gather(prows_v, [uids, cold])
        pd = pd + uv * pv
        acc_u2 = acc_u2 + uv * uv
        acc_p2 = acc_p2 + pv * pv
      bv = plsc.load_gather(bpos_v, [zero16, uids])
      acc_bi2 = acc_bi2 + bv * bv
      plsc.store_scatter(pscore_v, [zero16, uids], T * pd + bv - 1.0)

    def chunk_body(c, accs):
      cps = []
      for j in range(GPC):
        g = c * GPC + j
        cps.append(pltpu.async_copy(qi_hbm.at[nidx_v.at[g]],
                                    nrows_v.at[pl.ds(j * 128, 128)], sem))
        cps.append(pltpu.async_copy(bi_hbm.at[nidx_v.at[g]],
                                    bneg_v.at[pl.ds(j * 128, 128)], sem))
      for cp in cps:
        cp.wait()

      def grp_body(g, accs2):
        acc_sq, acc_n2, acc_bj2 = accs2
        rows = lanes + g * 16
        p_local = c * ROWS_PC + g * 16 + lanes
        uids = p_local // NNEG
        dot = f32z
        an2 = f32z
        for d in range(D):
          cold = jnp.full((16,), d, jnp.int32)
          nv = plsc.load_gather(nrows_v, [rows, cold])
          ut = plsc.load_gather(urows_v, [uids, cold])
          dot = dot + ut * nv
          an2 = an2 + nv * nv
        bj = plsc.load_gather(bneg_v, [rows])
        cs = plsc.load_gather(pscore_v, [zero16, uids])
        diff = cs - (T * dot + bj)
        return (acc_sq + diff * diff, acc_n2 + an2, acc_bj2 + bj * bj)

      return lax.fori_loop(0, PG_PC, grp_body, accs)

    acc_sq, acc_n2, acc_bj2 = lax.fori_loop(
        0, NCHUNK, chunk_body, (f32z, f32z, f32z))

    total = (acc_sq * (1.0 / float(B * NNEG))
             + BATA * (acc_u2 + acc_p2 + acc_n2)
             + LAMDA * (acc_bi2 + acc_bj2))
    out_v[0, :] = total
    pltpu.sync_copy(out_v, out_hbm.at[w])

  return _sc_fused


def _tc_reduce_body(x_ref, out_ref):
  out_ref[0, 0] = jnp.sum(x_ref[...])


@functools.cache
def _build_tc_reduce():
  return pl.pallas_call(
      _tc_reduce_body,
      in_specs=[pl.BlockSpec((NW, 16), lambda: (0, 0))],
      out_specs=pl.BlockSpec(memory_space=pltpu.SMEM),
      out_shape=jax.ShapeDtypeStruct((1, 1), jnp.float32),
  )


def kernel(users, pos_items, neg_items, pu, qi, bi):
  users2 = users.reshape(NW, UPW)
  pos2 = pos_items.reshape(NW, UPW)
  neg3 = neg_items.reshape(NW, GPW, 128)
  bif = bi.reshape(-1)
  partials = _build_sc_fused()(users2, pos2, neg3, pu, qi, bif)
  res = _build_tc_reduce()(partials.reshape(NW, 16))
  return res[0, 0]
